# Initial kernel scaffold; baseline (speedup 1.0000x reference)
import jax
import jax.numpy as jnp
from jax.experimental import pallas as pl


def kernel(memory, input_logits):
    raise NotImplementedError("write your pallas kernel here")


# speedup vs baseline: 2.5041x; 2.5041x over previous
import jax
import jax.numpy as jnp
from jax.experimental import pallas as pl
from jax.experimental.pallas import tpu as pltpu


def kernel(memory, input_logits):
    size, dim = memory.shape
    num = input_logits.shape[0]
    block = num
    grid = (pl.cdiv(size, block),)

    def body(mem_ref, logits_ref, out_ref, idx_ref):
        i = pl.program_id(0)

        @pl.when(i == 0)
        def _():
            out_ref[...] = logits_ref[...]
            idx_ref[0] = jnp.int32(num % size)

        @pl.when(i > 0)
        def _():
            out_ref[...] = mem_ref[...]

    memory_new, new_index = pl.pallas_call(
        body,
        grid=grid,
        in_specs=[
            pl.BlockSpec((block, dim), lambda i: (i, 0)),
            pl.BlockSpec((num, dim), lambda i: (0, 0)),
        ],
        out_specs=[
            pl.BlockSpec((block, dim), lambda i: (i, 0)),
            pl.BlockSpec(memory_space=pltpu.SMEM),
        ],
        out_shape=[
            jax.ShapeDtypeStruct((size, dim), memory.dtype),
            jax.ShapeDtypeStruct((1,), jnp.int32),
        ],
    )(memory, input_logits)
    return (memory_new, new_index[0])
